# Initial kernel scaffold; baseline (speedup 1.0000x reference)
import jax
import jax.numpy as jnp
from jax.experimental import pallas as pl


def kernel(coords, features, kernel_weights, bias):
    raise NotImplementedError("write your pallas kernel here")


# speedup vs baseline: 11.6459x; 11.6459x over previous
import functools
import math

import jax
import jax.numpy as jnp
import numpy as np
from jax import lax
from jax.experimental import pallas as pl
from jax.experimental.pallas import tpu as pltpu
from jax.experimental.pallas import tpu_sc as plsc

NKP = 15
RADIUS = 1.0
SIGMA = 0.1
KNB = 16
RB = 128
CT = 512
NEG_INF_GUARD = -1.0
INF = float("inf")


def _kernel_points_np():
    pts = []
    offset = 2.0 / NKP
    increment = math.pi * (3.0 - math.sqrt(5.0))
    for i in range(NKP):
        y = i * offset - 1 + offset / 2
        r = math.sqrt(max(0.0, 1 - y * y))
        phi = i * increment
        pts.append([math.cos(phi) * r, y, math.sin(phi) * r])
    return np.asarray(pts, dtype=np.float32) * RADIUS


def _kpm_np():
    kp = _kernel_points_np()
    m = np.zeros((32, 16), dtype=np.float32)
    m[0:3, :NKP] = -2.0 * kp.T
    m[3, :NKP] = np.sum(kp * kp, axis=1)
    m[16:19, :NKP] = 1.0
    return m


def _extract16(av, at, pv, pc):
    lane = lax.broadcasted_iota(jnp.int32, (RB, 128), 1)
    acol = at * 128 + lane
    cv = jnp.concatenate([pv, av], axis=1)
    cc = jnp.concatenate([pc, acol], axis=1)
    sel_a = jnp.zeros((RB, 128), jnp.bool_)
    bigc = jnp.int32(2 ** 30)
    pvs, pcs = [], []
    for _ in range(KNB):
        rowmin = jnp.min(cv, axis=1, keepdims=True)
        eq = cv == rowmin
        colsel = jnp.min(jnp.where(eq, cc, bigc), axis=1, keepdims=True)
        selpos = eq & (cc == colsel)
        pvs.append(rowmin)
        pcs.append(colsel)
        cv = jnp.where(selpos, jnp.float32(INF), cv)
        sel_a = sel_a | selpos[:, KNB:]
    return (jnp.concatenate(pvs, axis=1), jnp.concatenate(pcs, axis=1), sel_a)


def _knn_body(xt_ref, xrow_ref, idx_ref, panel_ref, *, n_valid, np_pad):
    blk = pl.program_id(0)
    nct = np_pad // CT
    qpt = CT // 128

    xb_bf = xt_ref[:, pl.ds(blk * RB, RB)].astype(jnp.bfloat16)
    xbr = xrow_ref[...]
    sqb_col = jnp.sum(xbr * xbr, axis=1, keepdims=True)

    row_g = blk * RB + lax.broadcasted_iota(jnp.int32, (RB, CT), 0)

    def build_fold(t, carry):
        av, at = carry
        xt_t = xt_ref[:, pl.ds(t * CT, CT)]
        sq_t = jnp.sum(xt_t * xt_t, axis=0, keepdims=True)
        g = lax.dot_general(xb_bf, xt_t.astype(jnp.bfloat16),
                            (((0,), (0,)), ((), ())),
                            preferred_element_type=jnp.float32)
        d2 = (sqb_col + sq_t) - 2.0 * g
        d2 = jnp.maximum(d2, 0.0)
        col_g = t * CT + lax.broadcasted_iota(jnp.int32, (RB, CT), 1)
        d2 = jnp.where((col_g == row_g) | (col_g >= n_valid),
                       jnp.float32(INF), d2)
        panel_ref[:, pl.ds(t * CT, CT)] = d2
        for c in range(qpt):
            chunk = d2[:, c * 128:(c + 1) * 128]
            q = t * qpt + c
            upd = chunk < av
            av = jnp.where(upd, chunk, av)
            at = jnp.where(upd, q, at)
        return av, at

    av0 = jnp.full((RB, 128), INF, jnp.float32)
    at0 = jnp.zeros((RB, 128), jnp.int32)
    av, at = lax.fori_loop(0, nct, build_fold, (av0, at0))

    pv0 = jnp.full((RB, KNB), INF, jnp.float32)
    pc0 = jnp.full((RB, KNB), 2 ** 30, jnp.int32)
    pv, pc, sel_a = _extract16(av, at, pv0, pc0)
    dv = jnp.where(sel_a, av, jnp.float32(NEG_INF_GUARD))
    dt = jnp.where(sel_a, at, -1)
    done = jnp.sum(sel_a.astype(jnp.int32)) == 0

    def round_body(_, state):
        def do_round(state):
            pv, pc, dv, dt, done = state

            def refold(t, carry):
                av, at = carry
                tile = panel_ref[:, pl.ds(t * CT, CT)]
                dvt = jnp.concatenate([dv] * qpt, axis=1)
                dtt = jnp.concatenate([dt] * qpt, axis=1)
                lane = lax.broadcasted_iota(jnp.int32, (RB, CT), 1)
                qmat = t * qpt + lane // 128
                deleted = (tile < dvt) | ((tile == dvt) & (qmat <= dtt))
                tile = jnp.where(deleted, jnp.float32(INF), tile)
                for c in range(qpt):
                    chunk = tile[:, c * 128:(c + 1) * 128]
                    q = t * qpt + c
                    upd = chunk < av
                    av = jnp.where(upd, chunk, av)
                    at = jnp.where(upd, q, at)
                return av, at

            av, at = lax.fori_loop(0, nct, refold, (av0, at0))
            pv2, pc2, sel_a = _extract16(av, at, pv, pc)
            dv2 = jnp.where(sel_a, av, dv)
            dt2 = jnp.where(sel_a, at, dt)
            done2 = jnp.sum(sel_a.astype(jnp.int32)) == 0
            return pv2, pc2, dv2, dt2, done2

        return lax.cond(state[4], lambda s: s, do_round, state)

    pv, pc, dv, dt, done = lax.fori_loop(
        0, KNB - 1, round_body, (pv, pc, dv, dt, done))
    idx_ref[...] = jnp.clip(pc, 0, n_valid - 1)


def _knn_topk(xt_pad, n_valid, np_pad):
    nb = np_pad // RB
    xrow = jnp.pad(jnp.transpose(xt_pad), ((0, 0), (0, 5)))
    return pl.pallas_call(
        functools.partial(_knn_body, n_valid=n_valid, np_pad=np_pad),
        grid=(nb,),
        in_specs=[pl.BlockSpec((3, np_pad), lambda i: (0, 0)),
                  pl.BlockSpec((RB, 8), lambda i: (i, 0))],
        out_specs=pl.BlockSpec((RB, KNB), lambda i: (i, 0)),
        out_shape=jax.ShapeDtypeStruct((np_pad, KNB), jnp.int32),
        scratch_shapes=[pltpu.VMEM((RB, np_pad), jnp.float32)],
    )(xt_pad, xrow)


def _sc_gather(feat_t, coord_t, idx_flat, np_pad):
    ne = idx_flat.shape[0]
    info = plsc.get_sparse_core_info()
    nw = info.num_cores * info.num_subcores
    b_per_w = ne // nw
    ch = 128
    nch = b_per_w // ch
    mesh = plsc.VectorSubcoreMesh(core_axis_name="c", subcore_axis_name="s")

    @functools.partial(
        pl.kernel, mesh=mesh,
        out_type=[jax.ShapeDtypeStruct((ne, 128), jnp.float32),
                  jax.ShapeDtypeStruct((ne, 128), jnp.float32)],
        scratch_types=[pltpu.VMEM((ch,), jnp.int32),
                       pltpu.VMEM((ch, 128), jnp.float32),
                       pltpu.VMEM((ch, 128), jnp.float32),
                       pltpu.SemaphoreType.DMA,
                       pltpu.SemaphoreType.DMA],
    )
    def gather(tf_hbm, tc_hbm, idx_hbm, outf_hbm, outc_hbm,
               idx_v, rf_v, rc_v, sem_f, sem_c):
        wid = lax.axis_index("s") * info.num_cores + lax.axis_index("c")
        base = wid * b_per_w

        def body(c, carry):
            off = base + c * ch
            pltpu.sync_copy(idx_hbm.at[pl.ds(off, ch)], idx_v)
            cpf = pltpu.async_copy(tf_hbm.at[idx_v], rf_v, sem_f)
            cpc = pltpu.async_copy(tc_hbm.at[idx_v], rc_v, sem_c)
            cpf.wait()
            cpc.wait()
            pltpu.sync_copy(rf_v, outf_hbm.at[pl.ds(off, ch)])
            pltpu.sync_copy(rc_v, outc_hbm.at[pl.ds(off, ch)])
            return carry

        lax.fori_loop(0, nch, body, 0)

    return gather(feat_t, coord_t, idx_flat)


def _agg_body(nf_ref, nc_ref, xa_ref, kpm_ref, w2_ref, bias_ref, out_ref):
    pb = RB
    nf = nf_ref[...]
    nc = nc_ref[:, :16]
    xa = xa_ref[...]

    e_iota = lax.broadcasted_iota(jnp.int32, (pb * KNB, pb), 0) // KNB
    p_iota = lax.broadcasted_iota(jnp.int32, (pb * KNB, pb), 1)
    r_mat = (e_iota == p_iota).astype(jnp.float32)
    xrep = jnp.dot(r_mat, xa, precision=lax.Precision.HIGHEST,
                   preferred_element_type=jnp.float32)

    de = nc - xrep
    lane16 = lax.broadcasted_iota(jnp.int32, (pb * KNB, 16), 1)
    aug = jnp.where(lane16 == 3, 1.0, de)
    x2 = jnp.concatenate([aug, de * de], axis=1)
    sqd = jnp.dot(x2, kpm_ref[...], precision=lax.Precision.HIGHEST,
                  preferred_element_type=jnp.float32)
    infl = jnp.exp(sqd * (-1.0 / (SIGMA * SIGMA)))

    infl3 = infl.reshape(pb, KNB, 16)
    nf3 = nf.reshape(pb, KNB, 128)
    agg = jnp.zeros((pb, 16, 128), jnp.float32)
    for j in range(KNB):
        agg = agg + infl3[:, j, :, None] * nf3[:, j, None, :]

    acc = jnp.zeros((pb, 128), jnp.float32) + bias_ref[...]
    for k in range(NKP):
        acc = acc + jnp.dot(agg[:, k, :], w2_ref[k],
                            precision=lax.Precision.HIGHEST,
                            preferred_element_type=jnp.float32)
    out_ref[...] = acc


def _kpconv_agg(nf, nc, xa, kpm, w2, bias2d, np_pad):
    nb = np_pad // RB
    return pl.pallas_call(
        _agg_body,
        grid=(nb,),
        in_specs=[
            pl.BlockSpec((RB * KNB, 128), lambda i: (i, 0)),
            pl.BlockSpec((RB * KNB, 128), lambda i: (i, 0)),
            pl.BlockSpec((RB, 16), lambda i: (i, 0)),
            pl.BlockSpec((32, 16), lambda i: (0, 0)),
            pl.BlockSpec((NKP, 128, 128), lambda i: (0, 0, 0)),
            pl.BlockSpec((1, 128), lambda i: (0, 0)),
        ],
        out_specs=pl.BlockSpec((RB, 128), lambda i: (i, 0)),
        out_shape=jax.ShapeDtypeStruct((np_pad, 128), jnp.float32),
    )(nf, nc, xa, kpm, w2, bias2d)


def kernel(coords, features, kernel_weights, bias):
    b, _, n = coords.shape
    np_pad = ((n + CT - 1) // CT) * CT

    xt = coords[0]
    xt_pad = jnp.pad(xt, ((0, 0), (0, np_pad - n)))

    idx2d = _knn_topk(xt_pad, n, np_pad)
    idx_flat = idx2d.reshape(np_pad * KNB)

    feat_t = jnp.pad(features[0].T, ((0, np_pad - n), (0, 0)))
    coord_w = jnp.pad(xt.T, ((0, np_pad - n), (0, 125)))
    coord_t = coord_w[:, :16]

    nf, nc = _sc_gather(feat_t, coord_w, idx_flat, np_pad)

    kpm = jnp.asarray(_kpm_np())
    out = _kpconv_agg(nf, nc, coord_t, kpm, kernel_weights,
                      bias.reshape(1, -1), np_pad)
    return out[:n].T[None]
